# baseline (speedup 1.0000x reference)
import functools

import jax
import jax.numpy as jnp
from jax import lax
from jax.experimental import pallas as pl
from jax.experimental.pallas import tpu as pltpu
from jax.experimental.pallas import tpu_sc as plsc

VOCAB = 5
EMBED_DIM = 256
MAX_LEN = 512
EPS = 1e-5

BATCH_BLK = 32


def _combine_kernel(tok_ref, pos_ref, gamma_ref, beta_ref, out_ref):
    emb = tok_ref[...][:, None, :] + pos_ref[...][None, :, :]
    mean = jnp.mean(emb, axis=-1, keepdims=True)
    var = jnp.mean(jnp.square(emb - mean), axis=-1, keepdims=True)
    normed = (emb - mean) * jax.lax.rsqrt(var + EPS)
    out_ref[...] = normed * gamma_ref[...][None, None, :] + beta_ref[...][None, None, :]


def _expand_kernel(x_ref, comb_ref, out_ref):
    xb = x_ref[...]
    c = comb_ref[...]
    sel = xb[:, :, None]
    r = jnp.where(sel == 0, c[0][None], c[4][None])
    r = jnp.where(sel == 1, c[1][None], r)
    r = jnp.where(sel == 2, c[2][None], r)
    r = jnp.where(sel == 3, c[3][None], r)
    out_ref[...] = r


def _idx_kernel(x_ref, idx_ref):
    s_iota = lax.broadcasted_iota(jnp.int32, x_ref.shape, 1)
    idx_ref[...] = x_ref[...] * MAX_LEN + s_iota


NUM_WORKERS = 32
SC_CHUNK = 128


def _sc_expand(total_rows, dim):
    b_per_w = total_rows // NUM_WORKERS
    n_chunks = b_per_w // SC_CHUNK
    mesh = plsc.VectorSubcoreMesh(core_axis_name="c", subcore_axis_name="s")

    @functools.partial(
        pl.kernel,
        mesh=mesh,
        out_type=jax.ShapeDtypeStruct((total_rows, dim), jnp.float32),
        scratch_types=[
            pltpu.VMEM((b_per_w,), jnp.int32),
            pltpu.VMEM((SC_CHUNK, dim), jnp.float32),
            pltpu.SemaphoreType.DMA,
        ],
    )
    def expand(idx_hbm, table_hbm, out_hbm, idx_v, rows_v, sem):
        wid = lax.axis_index("s") * 2 + lax.axis_index("c")
        base = wid * b_per_w
        pltpu.sync_copy(idx_hbm.at[pl.ds(base, b_per_w)], idx_v)

        def body(c, carry):
            pltpu.async_copy(
                table_hbm.at[idx_v.at[pl.ds(c * SC_CHUNK, SC_CHUNK)]],
                rows_v, sem).wait()
            pltpu.sync_copy(rows_v, out_hbm.at[pl.ds(base + c * SC_CHUNK, SC_CHUNK)])
            return carry

        lax.fori_loop(0, n_chunks, body, 0)

    return expand


@functools.partial(jax.jit, static_argnums=())
def kernel(x, token_table, pos_table, gamma, beta):
    batch, seq = x.shape
    vocab, dim = token_table.shape

    combined = pl.pallas_call(
        _combine_kernel,
        out_shape=jax.ShapeDtypeStruct((vocab, seq, dim), jnp.float32),
    )(token_table, pos_table[:seq], gamma, beta)

    x = x.astype(jnp.int32)
    idx = pl.pallas_call(
        _idx_kernel,
        out_shape=jax.ShapeDtypeStruct((batch, seq), jnp.int32),
    )(x)
    out = _sc_expand(batch * seq, dim)(
        idx.reshape(batch * seq), combined.reshape(vocab * seq, dim))
    return out.reshape(batch, seq, dim)


# speedup vs baseline: 1.1507x; 1.1507x over previous
import functools

import jax
import jax.numpy as jnp
from jax import lax
from jax.experimental import pallas as pl
from jax.experimental.pallas import tpu as pltpu
from jax.experimental.pallas import tpu_sc as plsc

VOCAB = 5
EMBED_DIM = 256
MAX_LEN = 512
EPS = 1e-5

BATCH_BLK = 32


def _combine_kernel(tok_ref, pos_ref, gamma_ref, beta_ref, out_ref):
    emb = tok_ref[...][:, None, :] + pos_ref[...][None, :, :]
    mean = jnp.mean(emb, axis=-1, keepdims=True)
    var = jnp.mean(jnp.square(emb - mean), axis=-1, keepdims=True)
    normed = (emb - mean) * jax.lax.rsqrt(var + EPS)
    out_ref[...] = normed * gamma_ref[...][None, None, :] + beta_ref[...][None, None, :]


def _expand_kernel(x_ref, comb_ref, out_ref):
    xb = x_ref[...]
    c = comb_ref[...]
    sel = xb[:, :, None]
    r = jnp.where(sel == 0, c[0][None], c[4][None])
    r = jnp.where(sel == 1, c[1][None], r)
    r = jnp.where(sel == 2, c[2][None], r)
    r = jnp.where(sel == 3, c[3][None], r)
    out_ref[...] = r


def _idx_kernel(x_ref, idx_ref):
    s_iota = lax.broadcasted_iota(jnp.int32, x_ref.shape, 1)
    idx_ref[...] = x_ref[...] * MAX_LEN + s_iota


NUM_WORKERS = 32
SC_CHUNK = 128


def _sc_expand(total_rows, dim):
    b_per_w = total_rows // NUM_WORKERS
    n_chunks = b_per_w // SC_CHUNK
    mesh = plsc.VectorSubcoreMesh(core_axis_name="c", subcore_axis_name="s")

    @functools.partial(
        pl.kernel,
        mesh=mesh,
        out_type=jax.ShapeDtypeStruct((total_rows, dim), jnp.float32),
        scratch_types=[
            pltpu.VMEM((b_per_w,), jnp.int32),
            pltpu.VMEM((SC_CHUNK, dim), jnp.float32),
            pltpu.VMEM((SC_CHUNK, dim), jnp.float32),
            pltpu.SemaphoreType.DMA,
            pltpu.SemaphoreType.DMA,
            pltpu.SemaphoreType.DMA,
            pltpu.SemaphoreType.DMA,
        ],
    )
    def expand(idx_hbm, table_hbm, out_hbm, idx_v, rows0, rows1,
               g0, g1, s0, s1):
        wid = lax.axis_index("s") * 2 + lax.axis_index("c")
        base = wid * b_per_w
        pltpu.sync_copy(idx_hbm.at[pl.ds(base, b_per_w)], idx_v)
        bufs = (rows0, rows1)
        gsems = (g0, g1)
        ssems = (s0, s1)

        def gather(c, b):
            return pltpu.make_async_copy(
                table_hbm.at[idx_v.at[pl.ds(c * SC_CHUNK, SC_CHUNK)]],
                bufs[b], gsems[b])

        def scatter(c, b):
            return pltpu.make_async_copy(
                bufs[b], out_hbm.at[pl.ds(base + c * SC_CHUNK, SC_CHUNK)],
                ssems[b])

        gather(0, 0).start()

        def body(c2, _):
            for b in range(2):
                c = c2 * 2 + b
                gather(c, b).wait()
                nxt = c + 1
                nb = 1 - b

                @pl.when(c >= 1)
                def _():
                    scatter(c - 1, nb).wait()

                @pl.when(nxt < n_chunks)
                def _():
                    gather(nxt, nb).start()

                scatter(c, b).start()
            return 0

        lax.fori_loop(0, n_chunks // 2, body, 0)
        scatter(n_chunks - 1, (n_chunks - 1) % 2).wait()

    return expand


@functools.partial(jax.jit, static_argnums=())
def kernel(x, token_table, pos_table, gamma, beta):
    batch, seq = x.shape
    vocab, dim = token_table.shape

    combined = pl.pallas_call(
        _combine_kernel,
        out_shape=jax.ShapeDtypeStruct((vocab, seq, dim), jnp.float32),
    )(token_table, pos_table[:seq], gamma, beta)

    x = x.astype(jnp.int32)
    idx = pl.pallas_call(
        _idx_kernel,
        out_shape=jax.ShapeDtypeStruct((batch, seq), jnp.int32),
    )(x)
    out = _sc_expand(batch * seq, dim)(
        idx.reshape(batch * seq), combined.reshape(vocab * seq, dim))
    return out.reshape(batch, seq, dim)


# speedup vs baseline: 1.1794x; 1.0249x over previous
import functools

import jax
import jax.numpy as jnp
from jax import lax
from jax.experimental import pallas as pl
from jax.experimental.pallas import tpu as pltpu
from jax.experimental.pallas import tpu_sc as plsc

VOCAB = 5
EMBED_DIM = 256
MAX_LEN = 512
EPS = 1e-5

BATCH_BLK = 32


def _combine_kernel(tok_ref, pos_ref, gamma_ref, beta_ref, out_ref):
    emb = tok_ref[...][:, None, :] + pos_ref[...][None, :, :]
    mean = jnp.mean(emb, axis=-1, keepdims=True)
    var = jnp.mean(jnp.square(emb - mean), axis=-1, keepdims=True)
    normed = (emb - mean) * jax.lax.rsqrt(var + EPS)
    out_ref[...] = normed * gamma_ref[...][None, None, :] + beta_ref[...][None, None, :]


def _expand_kernel(x_ref, comb_ref, out_ref):
    xb = x_ref[...]
    c = comb_ref[...]
    sel = xb[:, :, None]
    r = jnp.where(sel == 0, c[0][None], c[4][None])
    r = jnp.where(sel == 1, c[1][None], r)
    r = jnp.where(sel == 2, c[2][None], r)
    r = jnp.where(sel == 3, c[3][None], r)
    out_ref[...] = r


def _idx_kernel(x_ref, idx_ref):
    s_iota = lax.broadcasted_iota(jnp.int32, x_ref.shape, 1)
    idx_ref[...] = x_ref[...] * MAX_LEN + s_iota


NUM_WORKERS = 32
SC_CHUNK = 64


NBUF = 4
LOOKAHEAD = 2


def _sc_expand(total_rows, dim):
    b_per_w = total_rows // NUM_WORKERS
    n_chunks = b_per_w // SC_CHUNK
    mesh = plsc.VectorSubcoreMesh(core_axis_name="c", subcore_axis_name="s")

    @functools.partial(
        pl.kernel,
        mesh=mesh,
        out_type=jax.ShapeDtypeStruct((total_rows, dim), jnp.float32),
        scratch_types=(
            [pltpu.VMEM((b_per_w,), jnp.int32)]
            + [pltpu.VMEM((SC_CHUNK, dim), jnp.float32)] * NBUF
            + [pltpu.SemaphoreType.DMA] * (2 * NBUF)
        ),
    )
    def expand(idx_hbm, table_hbm, out_hbm, idx_v, *bufs_sems):
        bufs = bufs_sems[:NBUF]
        gsems = bufs_sems[NBUF:2 * NBUF]
        ssems = bufs_sems[2 * NBUF:]
        wid = lax.axis_index("s") * 2 + lax.axis_index("c")
        base = wid * b_per_w
        pltpu.sync_copy(idx_hbm.at[pl.ds(base, b_per_w)], idx_v)

        def gather(c, b):
            return pltpu.make_async_copy(
                table_hbm.at[idx_v.at[pl.ds(c * SC_CHUNK, SC_CHUNK)]],
                bufs[b], gsems[b])

        def scatter(c, b):
            return pltpu.make_async_copy(
                bufs[b], out_hbm.at[pl.ds(base + c * SC_CHUNK, SC_CHUNK)],
                ssems[b])

        for p in range(LOOKAHEAD):
            gather(p, p % NBUF).start()

        def body(c2, _):
            for u in range(NBUF):
                c = c2 * NBUF + u
                b = u
                nxt = c + LOOKAHEAD
                nb = (u + LOOKAHEAD) % NBUF

                @pl.when(jnp.logical_and(nxt < n_chunks, c >= NBUF - LOOKAHEAD))
                def _():
                    scatter(nxt - NBUF, nb).wait()

                @pl.when(nxt < n_chunks)
                def _():
                    gather(nxt, nb).start()

                gather(c, b).wait()
                scatter(c, b).start()
            return 0

        lax.fori_loop(0, n_chunks // NBUF, body, 0)
        for u in range(NBUF):
            scatter(n_chunks - NBUF + u, u).wait()

    return expand


@functools.partial(jax.jit, static_argnums=())
def kernel(x, token_table, pos_table, gamma, beta):
    batch, seq = x.shape
    vocab, dim = token_table.shape

    combined = pl.pallas_call(
        _combine_kernel,
        out_shape=jax.ShapeDtypeStruct((vocab, seq, dim), jnp.float32),
    )(token_table, pos_table[:seq], gamma, beta)

    x = x.astype(jnp.int32)
    idx = pl.pallas_call(
        _idx_kernel,
        out_shape=jax.ShapeDtypeStruct((batch, seq), jnp.int32),
    )(x)
    out = _sc_expand(batch * seq, dim)(
        idx.reshape(batch * seq), combined.reshape(vocab * seq, dim))
    return out.reshape(batch, seq, dim)
